# baseline (speedup 1.0000x reference)
import functools
import math

import jax
import jax.numpy as jnp
import numpy as np
from jax import lax
from jax.experimental import pallas as pl
from jax.experimental.pallas import tpu as pltpu
from jax.experimental.pallas import tpu_sc as plsc

VOCAB = 1000000
HIDDEN = 64
MAX_LEN = 512
BATCH = 4096
SEQ = 200

NC = 2
NS = 16
NW = NC * NS
ROWS = BATCH * SEQ
RPW = ROWS // NW
SEQ_PER_W = RPW // SEQ
LANES = 16
VECS_PER_ROW = HIDDEN // LANES


def _make_pe_np(hidden_size=HIDDEN, max_len=MAX_LEN):
    position = np.arange(0, max_len, dtype=np.float32)[:, None]
    div_term = np.exp(
        np.arange(0, hidden_size, 2, dtype=np.float32)
        * (-math.log(10000.0) / hidden_size)
    )
    pe = np.zeros((max_len, hidden_size), dtype=np.float32)
    pe[:, 0::2] = np.sin(position * div_term)
    pe[:, 1::2] = np.cos(position * div_term)
    return pe


_PE = _make_pe_np()[:SEQ]


def _sc_embed(idx_flat, table, pe):
    mesh = plsc.VectorSubcoreMesh(core_axis_name="c", subcore_axis_name="s")

    @functools.partial(
        pl.kernel,
        out_type=jax.ShapeDtypeStruct((ROWS, HIDDEN), jnp.float32),
        mesh=mesh,
        compiler_params=pltpu.CompilerParams(use_tc_tiling_on_sc=False),
        scratch_types=[
            pltpu.VMEM((RPW,), jnp.int32),
            pltpu.VMEM((SEQ, HIDDEN), jnp.float32),
            pltpu.VMEM((SEQ, HIDDEN), jnp.float32),
            pltpu.SemaphoreType.DMA,
        ],
    )
    def k(idx_hbm, table_hbm, pe_hbm, out_hbm, idx_v, pe_v, rows_v, sem):
        wid = lax.axis_index("s") * NC + lax.axis_index("c")
        base = wid * RPW
        pltpu.sync_copy(idx_hbm.at[pl.ds(base, RPW)], idx_v)
        pltpu.sync_copy(pe_hbm, pe_v)

        def seq_body(s, carry):
            pltpu.async_copy(
                table_hbm.at[idx_v.at[pl.ds(s * SEQ, SEQ)]], rows_v, sem
            ).wait()

            def add_row(r, carry2):
                for c in range(VECS_PER_ROW):
                    sl = pl.ds(c * LANES, LANES)
                    rows_v[r, sl] = rows_v[r, sl] + pe_v[r, sl]
                return carry2

            lax.fori_loop(0, SEQ, add_row, 0)
            pltpu.sync_copy(rows_v, out_hbm.at[pl.ds(base + s * SEQ, SEQ)])
            return carry

        lax.fori_loop(0, SEQ_PER_W, seq_body, 0)

    return k(idx_flat, table, pe)


def kernel(input_ids, table):
    idx_flat = input_ids.reshape(-1).astype(jnp.int32)
    out = _sc_embed(idx_flat, table, jnp.asarray(_PE))
    return out.reshape(BATCH, SEQ, HIDDEN)


# speedup vs baseline: 1.1257x; 1.1257x over previous
import functools
import math

import jax
import jax.numpy as jnp
import numpy as np
from jax import lax
from jax.experimental import pallas as pl
from jax.experimental.pallas import tpu as pltpu
from jax.experimental.pallas import tpu_sc as plsc

VOCAB = 1000000
HIDDEN = 64
MAX_LEN = 512
BATCH = 4096
SEQ = 200

NC = 2
NS = 16
NW = NC * NS
ROWS = BATCH * SEQ
RPW = ROWS // NW
SEQ_PER_W = RPW // SEQ
LANES = 16
VECS_PER_ROW = HIDDEN // LANES

NBUF = 4
NCH = SEQ_PER_W
GROUPS = NCH // NBUF


def _make_pe_np(hidden_size=HIDDEN, max_len=MAX_LEN):
    position = np.arange(0, max_len, dtype=np.float32)[:, None]
    div_term = np.exp(
        np.arange(0, hidden_size, 2, dtype=np.float32)
        * (-math.log(10000.0) / hidden_size)
    )
    pe = np.zeros((max_len, hidden_size), dtype=np.float32)
    pe[:, 0::2] = np.sin(position * div_term)
    pe[:, 1::2] = np.cos(position * div_term)
    return pe


_PE = _make_pe_np()[:SEQ]


def _sc_embed(idx_flat, table, pe):
    mesh = plsc.VectorSubcoreMesh(core_axis_name="c", subcore_axis_name="s")

    @functools.partial(
        pl.kernel,
        out_type=jax.ShapeDtypeStruct((ROWS, HIDDEN), jnp.float32),
        mesh=mesh,
        compiler_params=pltpu.CompilerParams(use_tc_tiling_on_sc=False),
        scratch_types=(
            [pltpu.VMEM((RPW,), jnp.int32)]
            + [pltpu.VMEM((SEQ, HIDDEN), jnp.float32)]
            + [pltpu.VMEM((SEQ, HIDDEN), jnp.float32)] * NBUF
            + [pltpu.SemaphoreType.DMA] * (2 * NBUF)
        ),
    )
    def k(idx_hbm, table_hbm, pe_hbm, out_hbm, idx_v, pe_v, *rest):
        bufs = rest[:NBUF]
        gsem = rest[NBUF:2 * NBUF]
        osem = rest[2 * NBUF:]
        wid = lax.axis_index("s") * NC + lax.axis_index("c")
        base = wid * RPW
        pltpu.sync_copy(idx_hbm.at[pl.ds(base, RPW)], idx_v)
        pltpu.sync_copy(pe_hbm, pe_v)

        for b in range(NBUF):
            pltpu.async_copy(
                table_hbm.at[idx_v.at[pl.ds(b * SEQ, SEQ)]], bufs[b], gsem[b]
            )

        def group(g, carry):
            for b in range(NBUF):
                s = g * NBUF + b
                pltpu.make_async_copy(
                    table_hbm.at[idx_v.at[pl.ds(s * SEQ, SEQ)]], bufs[b], gsem[b]
                ).wait()

                @plsc.parallel_loop(0, SEQ, unroll=8)
                def add_row(r):
                    for c in range(VECS_PER_ROW):
                        sl = pl.ds(c * LANES, LANES)
                        bufs[b][r, sl] = bufs[b][r, sl] + pe_v[r, sl]

                wb = pltpu.async_copy(
                    bufs[b], out_hbm.at[pl.ds(base + s * SEQ, SEQ)], osem[b]
                )

                @pl.when(g < GROUPS - 1)
                def _():
                    wb.wait()
                    pltpu.async_copy(
                        table_hbm.at[idx_v.at[pl.ds((s + NBUF) * SEQ, SEQ)]],
                        bufs[b],
                        gsem[b],
                    )

            return carry

        lax.fori_loop(0, GROUPS, group, 0)

        for b in range(NBUF):
            pltpu.make_async_copy(
                bufs[b], out_hbm.at[pl.ds(base, SEQ)], osem[b]
            ).wait()

    return k(idx_flat, table, pe)


def kernel(input_ids, table):
    idx_flat = input_ids.reshape(-1).astype(jnp.int32)
    out = _sc_embed(idx_flat, table, jnp.asarray(_PE))
    return out.reshape(BATCH, SEQ, HIDDEN)
